# Initial kernel scaffold; baseline (speedup 1.0000x reference)
import jax
import jax.numpy as jnp
from jax.experimental import pallas as pl


def kernel(input, input_lengths, emb_table, lin_w):
    raise NotImplementedError("write your pallas kernel here")


# speedup vs baseline: 13.1452x; 13.1452x over previous
import functools

import jax
import jax.numpy as jnp
from jax import lax
from jax.experimental import pallas as pl
from jax.experimental.pallas import tpu as pltpu
from jax.experimental.pallas import tpu_sc as plsc

NUM_EMB = 1_000_000
DIM = 32
BATCH = 16384
HIST = 50

_TC_BLK = 32768
_T_PAD = 31 * _TC_BLK


def _matvec_body(tbl_ref, w_ref, out_ref):
    prod = tbl_ref[...] * w_ref[...]
    out_ref[...] = jnp.sum(prod, axis=0)


def _table_matvec(emb_t, w_col):
    return pl.pallas_call(
        _matvec_body,
        grid=(_T_PAD // _TC_BLK,),
        in_specs=[
            pl.BlockSpec((DIM, _TC_BLK), lambda i: (0, i)),
            pl.BlockSpec((DIM, 1), lambda i: (0, 0)),
        ],
        out_specs=pl.BlockSpec((_TC_BLK,), lambda i: (i,)),
        out_shape=jax.ShapeDtypeStruct((_T_PAD,), jnp.float32),
    )(emb_t, w_col)


_NW = 32
_COLS_W = BATCH // _NW
_CHUNK = 128
_NCHUNK = _COLS_W * HIST // _CHUNK
_FIRE = 8


_IDX_W = _COLS_W * HIST


def _sc_body(idx_hbm, t_hbm, out_hbm, idx_v, vals_v, out_v, sem):
    wid = lax.axis_index("s") * 2 + lax.axis_index("c")

    pltpu.sync_copy(idx_hbm.at[wid], idx_v)

    def _copy(j):
        return pltpu.make_async_copy(
            t_hbm.at[idx_v.at[j]],
            vals_v.at[pl.ds(j * _CHUNK, _CHUNK)],
            sem,
        )

    def _gather_chunk(b, carry):
        base = b * _FIRE
        for k in range(_FIRE):
            _copy(base + k).start()
        for k in range(_FIRE):
            _copy(base + k).wait()
        return carry

    lax.fori_loop(0, _NCHUNK // _FIRE, _gather_chunk, 0)

    def _reduce_group(g, carry):
        col = g * 16
        acc = jnp.zeros((16,), jnp.float32)
        for l in range(HIST):
            acc = acc + vals_v[pl.ds(l * _COLS_W + col, 16)]
        out_v[pl.ds(col, 16)] = acc
        return carry

    lax.fori_loop(0, _COLS_W // 16, _reduce_group, 0)

    pltpu.sync_copy(out_v, out_hbm.at[pl.ds(wid * _COLS_W, _COLS_W)])


@functools.partial(
    pl.kernel,
    mesh=plsc.VectorSubcoreMesh(core_axis_name="c", subcore_axis_name="s"),
    out_type=jax.ShapeDtypeStruct((BATCH,), jnp.float32),
    scratch_types=[
        pltpu.VMEM((_NCHUNK, _CHUNK), jnp.int32),
        pltpu.VMEM((_IDX_W,), jnp.float32),
        pltpu.VMEM((_COLS_W,), jnp.float32),
        pltpu.SemaphoreType.DMA,
    ],
    compiler_params=pltpu.CompilerParams(needs_layout_passes=False),
)
def _sc_gather_sum(idx_hbm, t_hbm, out_hbm, idx_v, vals_v, out_v, sem):
    _sc_body(idx_hbm, t_hbm, out_hbm, idx_v, vals_v, out_v, sem)


def kernel(input, input_lengths, emb_table, lin_w):
    del input_lengths
    t = _table_matvec(emb_table.T, lin_w.T)
    idx_arr = (
        input.T.astype(jnp.int32)
        .reshape(HIST, _NW, _COLS_W)
        .transpose(1, 0, 2)
        .reshape(_NW, _NCHUNK, _CHUNK)
    )
    out = _sc_gather_sum(idx_arr, t)
    return out.reshape(BATCH, 1)


# speedup vs baseline: 15.5400x; 1.1822x over previous
import functools

import jax
import jax.numpy as jnp
from jax import lax
from jax.experimental import pallas as pl
from jax.experimental.pallas import tpu as pltpu
from jax.experimental.pallas import tpu_sc as plsc

NUM_EMB = 1_000_000
DIM = 32
BATCH = 16384
HIST = 50

_TC_BLK = 32768
_T_PAD = 31 * _TC_BLK


def _matvec_body(tbl_ref, w_ref, out_ref):
    prod = tbl_ref[...] * w_ref[...]
    out_ref[...] = jnp.sum(prod, axis=0)


def _table_matvec(emb_t, w_col):
    return pl.pallas_call(
        _matvec_body,
        grid=(_T_PAD // _TC_BLK,),
        in_specs=[
            pl.BlockSpec((DIM, _TC_BLK), lambda i: (0, i)),
            pl.BlockSpec((DIM, 1), lambda i: (0, 0)),
        ],
        out_specs=pl.BlockSpec((_TC_BLK,), lambda i: (i,)),
        out_shape=jax.ShapeDtypeStruct((_T_PAD,), jnp.float32),
    )(emb_t, w_col)


_NW = 32
_COLS_W = BATCH // _NW
_CHUNK = 128
_NCHUNK = _COLS_W * HIST // _CHUNK
_ROWS_STR = 25


_IDX_W = _COLS_W * HIST


def _sc_body(idx_hbm, t_hbm, out_hbm, idx_v, vals_v, out_v, sem):
    wid = lax.axis_index("s") * 2 + lax.axis_index("c")

    pltpu.sync_copy(idx_hbm.at[wid], idx_v)

    _cp = pltpu.make_async_copy(t_hbm.at[idx_v], vals_v, sem)
    _cp.start()
    _cp.wait()

    def _reduce_group(g, carry):
        col = g * 16
        acc = jnp.zeros((16,), jnp.float32)
        for l in range(HIST):
            acc = acc + vals_v[pl.ds(l * _COLS_W + col, 16)]
        out_v[pl.ds(col, 16)] = acc
        return carry

    lax.fori_loop(0, _COLS_W // 16, _reduce_group, 0)

    pltpu.sync_copy(out_v, out_hbm.at[pl.ds(wid * _COLS_W, _COLS_W)])


@functools.partial(
    pl.kernel,
    mesh=plsc.VectorSubcoreMesh(core_axis_name="c", subcore_axis_name="s"),
    out_type=jax.ShapeDtypeStruct((BATCH,), jnp.float32),
    scratch_types=[
        pltpu.VMEM((_IDX_W,), jnp.int32),
        pltpu.VMEM((_IDX_W,), jnp.float32),
        pltpu.VMEM((_COLS_W,), jnp.float32),
        pltpu.SemaphoreType.DMA,
    ],
    compiler_params=pltpu.CompilerParams(needs_layout_passes=False),
)
def _sc_gather_sum(idx_hbm, t_hbm, out_hbm, idx_v, vals_v, out_v, sem):
    _sc_body(idx_hbm, t_hbm, out_hbm, idx_v, vals_v, out_v, sem)


def kernel(input, input_lengths, emb_table, lin_w):
    del input_lengths
    t = _table_matvec(emb_table.T, lin_w.T)
    idx_arr = (
        input.T.astype(jnp.int32)
        .reshape(HIST, _NW, _COLS_W)
        .transpose(1, 0, 2)
        .reshape(_NW, _IDX_W)
    )
    out = _sc_gather_sum(idx_arr, t)
    return out.reshape(BATCH, 1)
